# Initial kernel scaffold; baseline (speedup 1.0000x reference)
import jax
import jax.numpy as jnp
from jax.experimental import pallas as pl


def kernel(x, edge_index, W1, b1, W2, b2, W3, b3):
    raise NotImplementedError("write your pallas kernel here")


# speedup vs baseline: 14.8060x; 14.8060x over previous
import functools

import jax
import jax.numpy as jnp
from jax import lax
from jax.experimental import pallas as pl
from jax.experimental.pallas import tpu as pltpu
from jax.experimental.pallas import tpu_sc as plsc

NC = 2
NS = 16
NW = NC * NS
CH = 128
LN = 16


def _mesh():
    return plsc.VectorSubcoreMesh(
        core_axis_name="c", subcore_axis_name="s", num_cores=NC, num_subcores=NS
    )


@functools.cache
def _agg_kernel(n_pad, d, k_chunks, interpret=False):
    rpt = n_pad // NS
    ept = k_chunks * CH

    @functools.partial(
        pl.kernel,
        out_type=jax.ShapeDtypeStruct((NC * n_pad, d), jnp.float32),
        mesh=_mesh(),
        scratch_types=[
            pltpu.VMEM((CH,), jnp.int32),
            pltpu.VMEM((CH,), jnp.int32),
            pltpu.VMEM((CH, d), jnp.float32),
            pltpu.VMEM_SHARED((n_pad, d), jnp.float32),
            pltpu.SemaphoreType.DMA,
        ],
        compiler_params=pltpu.CompilerParams(use_tc_tiling_on_sc=False, needs_layout_passes=False),
        interpret=interpret,
    )
    def agg(u_hbm, src_hbm, dst_hbm, zero_hbm, out_hbm, sidx, didx, rows, acc, sem):
        c = lax.axis_index("c")
        s = lax.axis_index("s")
        wid = s * NC + c
        pltpu.sync_copy(zero_hbm.at[pl.ds(s * rpt, rpt)], acc.at[pl.ds(s * rpt, rpt)])
        plsc.subcore_barrier()
        base = wid * ept

        def step(j, carry):
            off = base + j * CH
            pltpu.sync_copy(src_hbm.at[pl.ds(off, CH)], sidx)
            pltpu.sync_copy(dst_hbm.at[pl.ds(off, CH)], didx)
            pltpu.async_copy(u_hbm.at[sidx], rows, sem).wait()
            pltpu.sync_copy(rows, acc.at[didx], add=True)
            return carry

        lax.fori_loop(0, k_chunks, step, 0)
        plsc.subcore_barrier()
        pltpu.sync_copy(
            acc.at[pl.ds(s * rpt, rpt)],
            out_hbm.at[pl.ds(c * n_pad + s * rpt, rpt)],
        )

    return agg


@functools.cache
def _score_kernel(e_pad, k_chunks, interpret=False):
    ept = k_chunks * CH

    @functools.partial(
        pl.kernel,
        out_type=jax.ShapeDtypeStruct((e_pad,), jnp.float32),
        mesh=_mesh(),
        scratch_types=[
            pltpu.VMEM((CH,), jnp.int32),
            pltpu.VMEM((CH,), jnp.int32),
            pltpu.VMEM((CH, LN), jnp.float32),
            pltpu.VMEM((CH, LN), jnp.float32),
            pltpu.VMEM((CH,), jnp.float32),
            pltpu.SemaphoreType.DMA,
            pltpu.SemaphoreType.DMA,
        ],
        compiler_params=pltpu.CompilerParams(use_tc_tiling_on_sc=False, needs_layout_passes=False),
        interpret=interpret,
    )
    def score(h_hbm, src_hbm, dst_hbm, out_hbm, sidx, didx, arows, brows, svec, sa, sb):
        c = lax.axis_index("c")
        s = lax.axis_index("s")
        wid = s * NC + c
        base = wid * ept

        def step(j, carry):
            off = base + j * CH
            pltpu.sync_copy(src_hbm.at[pl.ds(off, CH)], sidx)
            pltpu.sync_copy(dst_hbm.at[pl.ds(off, CH)], didx)
            ca = pltpu.async_copy(h_hbm.at[sidx], arows, sa)
            cb = pltpu.async_copy(h_hbm.at[didx], brows, sb)
            ca.wait()
            cb.wait()
            lane = lax.iota(jnp.int32, LN)
            for g in range(CH // LN):
                rid = lane + g * LN
                acc_v = jnp.zeros((LN,), jnp.float32)
                for dd in range(LN):
                    col = jnp.full((LN,), dd, jnp.int32)
                    acc_v = acc_v + plsc.load_gather(arows, [rid, col]) * plsc.load_gather(brows, [rid, col])
                svec[pl.ds(g * LN, LN)] = acc_v
            pltpu.sync_copy(svec, out_hbm.at[pl.ds(off, CH)])
            return carry

        lax.fori_loop(0, k_chunks, step, 0)

    return score


def _rows_spec(blk, d):
    return pl.BlockSpec((blk, d), lambda i: (i, 0))


def _gpart_spec(blk, d):
    return pl.BlockSpec((2, blk, d), lambda i: (0, i, 0))


def _full_spec(shape):
    return pl.BlockSpec(shape, lambda i: tuple(0 for _ in shape))


def _prep_body(degp_ref, x_ref, dinv_ref, u1_ref):
    deg = degp_ref[0, :, 0:1] + degp_ref[1, :, 0:1] + 1.0
    dv = lax.rsqrt(deg)
    dinv_ref[...] = dv
    u1_ref[...] = x_ref[...] * dv


def _l1_body(g_ref, u_ref, dinv_ref, w1_ref, b1_ref, w2_ref, out_ref):
    dv = dinv_ref[...]
    s1 = dv * (g_ref[0] + g_ref[1] + u_ref[...])
    h1 = jnp.maximum(
        jnp.dot(s1, w1_ref[...], preferred_element_type=jnp.float32) + b1_ref[...], 0.0
    )
    out_ref[...] = jnp.dot(h1, w2_ref[...], preferred_element_type=jnp.float32) * dv


def _l2_body(g_ref, u_ref, dinv_ref, b2_ref, w3_ref, out_ref):
    dv = dinv_ref[...]
    h2 = jnp.maximum(dv * (g_ref[0] + g_ref[1] + u_ref[...]) + b2_ref[...], 0.0)
    out_ref[...] = jnp.dot(h2, w3_ref[...], preferred_element_type=jnp.float32) * dv


def _l3_body(g_ref, u_ref, dinv_ref, b3_ref, out_ref):
    dv = dinv_ref[...]
    out_ref[...] = dv * (g_ref[0] + g_ref[1] + u_ref[...]) + b3_ref[...]


def kernel(x, edge_index, W1, b1, W2, b2, W3, b3):
    N = x.shape[0]
    E = edge_index.shape[1]
    n_pad = (N // 128 + 1) * 128
    k_chunks = -(-E // (NW * CH))
    e_pad = NW * CH * k_chunks
    blk = n_pad // 16
    grid = n_pad // blk

    f32 = jnp.float32
    src = jnp.concatenate([edge_index[0], jnp.zeros((e_pad - E,), jnp.int32)])
    dst = jnp.concatenate([edge_index[1], jnp.full((e_pad - E,), N, jnp.int32)])
    x8 = jnp.zeros((n_pad, 8), f32).at[:N, :3].set(x)
    ones8 = jnp.zeros((n_pad, 8), f32).at[:, 0].set(1.0)
    z8 = jnp.zeros((n_pad, 8), f32)
    z16 = jnp.zeros((n_pad, 16), f32)
    z32 = jnp.zeros((n_pad, 32), f32)
    W1p = jnp.zeros((8, 64), f32).at[:3].set(W1)
    b1r = b1.reshape(1, 64)
    b2r = b2.reshape(1, 32)
    b3r = b3.reshape(1, 16)

    agg8 = _agg_kernel(n_pad, 8, k_chunks)
    agg32 = _agg_kernel(n_pad, 32, k_chunks)
    agg16 = _agg_kernel(n_pad, 16, k_chunks)

    def parts(a, d):
        return a.reshape(2, n_pad, d)

    degp = parts(agg8(ones8, src, dst, z8), 8)

    dinv, u1 = pl.pallas_call(
        _prep_body,
        grid=(grid,),
        in_specs=[_gpart_spec(blk, 8), _rows_spec(blk, 8)],
        out_specs=[_rows_spec(blk, 1), _rows_spec(blk, 8)],
        out_shape=[
            jax.ShapeDtypeStruct((n_pad, 1), f32),
            jax.ShapeDtypeStruct((n_pad, 8), f32),
        ],
    )(degp, x8)

    g1 = parts(agg8(u1, src, dst, z8), 8)
    u2 = pl.pallas_call(
        _l1_body,
        grid=(grid,),
        in_specs=[
            _gpart_spec(blk, 8),
            _rows_spec(blk, 8),
            _rows_spec(blk, 1),
            _full_spec((8, 64)),
            _full_spec((1, 64)),
            _full_spec((64, 32)),
        ],
        out_specs=_rows_spec(blk, 32),
        out_shape=jax.ShapeDtypeStruct((n_pad, 32), f32),
    )(g1, u1, dinv, W1p, b1r, W2)

    g2 = parts(agg32(u2, src, dst, z32), 32)
    u3 = pl.pallas_call(
        _l2_body,
        grid=(grid,),
        in_specs=[
            _gpart_spec(blk, 32),
            _rows_spec(blk, 32),
            _rows_spec(blk, 1),
            _full_spec((1, 32)),
            _full_spec((32, 16)),
        ],
        out_specs=_rows_spec(blk, 16),
        out_shape=jax.ShapeDtypeStruct((n_pad, 16), f32),
    )(g2, u2, dinv, b2r, W3)

    g3 = parts(agg16(u3, src, dst, z16), 16)
    h3 = pl.pallas_call(
        _l3_body,
        grid=(grid,),
        in_specs=[
            _gpart_spec(blk, 16),
            _rows_spec(blk, 16),
            _rows_spec(blk, 1),
            _full_spec((1, 16)),
        ],
        out_specs=_rows_spec(blk, 16),
        out_shape=jax.ShapeDtypeStruct((n_pad, 16), f32),
    )(g3, u3, dinv, b3r)

    scores = _score_kernel(e_pad, k_chunks)(h3, src, dst)
    return scores[:E]


# speedup vs baseline: 22.4427x; 1.5158x over previous
import functools

import jax
import jax.numpy as jnp
from jax import lax
from jax.experimental import pallas as pl
from jax.experimental.pallas import tpu as pltpu
from jax.experimental.pallas import tpu_sc as plsc

NC = 2
NS = 16
NW = NC * NS
CH = 128
LN = 16


def _mesh():
    return plsc.VectorSubcoreMesh(
        core_axis_name="c", subcore_axis_name="s", num_cores=NC, num_subcores=NS
    )


NB = 8


@functools.cache
def _agg_kernel(n_pad, d, k_chunks, interpret=False):
    rpt = n_pad // NS
    nb = 4 if d >= 32 else NB
    kb = k_chunks // nb
    nsb = kb // 2

    @functools.partial(
        pl.kernel,
        out_type=jax.ShapeDtypeStruct((NC * n_pad, d), jnp.float32),
        mesh=_mesh(),
        scratch_types=[
            [pltpu.VMEM((nb, CH), jnp.int32) for _ in range(2)],
            [pltpu.VMEM((nb, CH), jnp.int32) for _ in range(2)],
            [pltpu.VMEM((CH, d), jnp.float32) for _ in range(nb)],
            pltpu.VMEM_SHARED((n_pad, d), jnp.float32),
            pltpu.SemaphoreType.DMA,
            pltpu.SemaphoreType.DMA,
            pltpu.SemaphoreType.DMA,
        ],
        compiler_params=pltpu.CompilerParams(use_tc_tiling_on_sc=False, needs_layout_passes=False),
        interpret=interpret,
    )
    def agg(u_hbm, src2_hbm, dst2_hbm, zero_hbm, out_hbm,
            sblk, dblk, rows, acc, gsem, ssem, isem):
        c = lax.axis_index("c")
        s = lax.axis_index("s")
        wid = s * NC + c
        pltpu.sync_copy(zero_hbm.at[pl.ds(s * rpt, rpt)], acc.at[pl.ds(s * rpt, rpt)])
        plsc.subcore_barrier()
        base = wid * kb

        def load_idx(blk_i, slot, sem):
            row0 = (base + blk_i) * nb
            a = pltpu.async_copy(src2_hbm.at[pl.ds(row0, nb)], sblk[slot], sem)
            b = pltpu.async_copy(dst2_hbm.at[pl.ds(row0, nb)], dblk[slot], sem)
            return a, b

        def half_block(slot):
            g = [pltpu.async_copy(u_hbm.at[sblk[slot].at[b]], rows[b], gsem)
                 for b in range(nb)]
            for x in g:
                x.wait()
            sc = [pltpu.async_copy(rows[b], acc.at[dblk[slot].at[b]], ssem, add=True)
                  for b in range(nb)]
            for x in sc:
                x.wait()

        for x in load_idx(0, 0, isem):
            x.wait()

        def super_block(p, carry):
            i1 = 2 * p + 1
            i2 = jnp.where(2 * p + 2 < kb, 2 * p + 2, 0)
            la, lb = load_idx(i1, 1, isem)
            half_block(0)
            la.wait()
            lb.wait()
            lc, ld = load_idx(i2, 0, isem)
            half_block(1)
            lc.wait()
            ld.wait()
            return carry

        lax.fori_loop(0, nsb, super_block, 0)
        plsc.subcore_barrier()
        pltpu.sync_copy(
            acc.at[pl.ds(s * rpt, rpt)],
            out_hbm.at[pl.ds(c * n_pad + s * rpt, rpt)],
        )

    return agg


@functools.cache
def _score_kernel(e_pad, k_chunks, interpret=False):
    kb = k_chunks // NB
    nsb = kb // 2

    @functools.partial(
        pl.kernel,
        out_type=jax.ShapeDtypeStruct((e_pad,), jnp.float32),
        mesh=_mesh(),
        scratch_types=[
            [pltpu.VMEM((NB, CH), jnp.int32) for _ in range(2)],
            [pltpu.VMEM((NB, CH), jnp.int32) for _ in range(2)],
            [pltpu.VMEM((CH, LN), jnp.float32) for _ in range(NB)],
            [pltpu.VMEM((CH, LN), jnp.float32) for _ in range(NB)],
            pltpu.VMEM((NB * CH,), jnp.float32),
            pltpu.SemaphoreType.DMA,
            pltpu.SemaphoreType.DMA,
        ],
        compiler_params=pltpu.CompilerParams(use_tc_tiling_on_sc=False, needs_layout_passes=False),
        interpret=interpret,
    )
    def score(h_hbm, src2_hbm, dst2_hbm, out_hbm,
              sblk, dblk, arows, brows, svec, gsem, isem):
        c = lax.axis_index("c")
        s = lax.axis_index("s")
        wid = s * NC + c
        base = wid * kb

        def load_idx(blk_i, slot, sem):
            row0 = (base + blk_i) * NB
            a = pltpu.async_copy(src2_hbm.at[pl.ds(row0, NB)], sblk[slot], sem)
            b = pltpu.async_copy(dst2_hbm.at[pl.ds(row0, NB)], dblk[slot], sem)
            return a, b

        def half_block(blk_i, slot):
            g = [pltpu.async_copy(h_hbm.at[sblk[slot].at[b]], arows[b], gsem)
                 for b in range(NB)]
            g += [pltpu.async_copy(h_hbm.at[dblk[slot].at[b]], brows[b], gsem)
                  for b in range(NB)]
            for x in g:
                x.wait()
            lane = lax.iota(jnp.int32, LN)
            for b in range(NB):
                for gg in range(CH // LN):
                    rid = lane + gg * LN
                    acc_v = jnp.zeros((LN,), jnp.float32)
                    for dd in range(LN):
                        col = jnp.full((LN,), dd, jnp.int32)
                        acc_v = acc_v + (plsc.load_gather(arows[b], [rid, col])
                                         * plsc.load_gather(brows[b], [rid, col]))
                    svec[pl.ds(b * CH + gg * LN, LN)] = acc_v
            pltpu.sync_copy(svec, out_hbm.at[pl.ds((base + blk_i) * NB * CH, NB * CH)])

        for x in load_idx(0, 0, isem):
            x.wait()

        def super_block(p, carry):
            i1 = 2 * p + 1
            i2 = jnp.where(2 * p + 2 < kb, 2 * p + 2, 0)
            la, lb = load_idx(i1, 1, isem)
            half_block(2 * p, 0)
            la.wait()
            lb.wait()
            lc, ld = load_idx(i2, 0, isem)
            half_block(i1, 1)
            lc.wait()
            ld.wait()
            return carry

        lax.fori_loop(0, nsb, super_block, 0)

    return score


def _rows_spec(blk, d):
    return pl.BlockSpec((blk, d), lambda i: (i, 0))


def _gpart_spec(blk, d):
    return pl.BlockSpec((2, blk, d), lambda i: (0, i, 0))


def _full_spec(shape):
    return pl.BlockSpec(shape, lambda i: tuple(0 for _ in shape))


def _prep_body(degp_ref, x_ref, dinv_ref, u1_ref):
    deg = degp_ref[0, :, 0:1] + degp_ref[1, :, 0:1] + 1.0
    dv = lax.rsqrt(deg)
    dinv_ref[...] = dv
    u1_ref[...] = x_ref[...] * dv


def _l1_body(g_ref, u_ref, dinv_ref, w1_ref, b1_ref, w2_ref, out_ref):
    dv = dinv_ref[...]
    s1 = dv * (g_ref[0] + g_ref[1] + u_ref[...])
    h1 = jnp.maximum(
        jnp.dot(s1, w1_ref[...], preferred_element_type=jnp.float32) + b1_ref[...], 0.0
    )
    out_ref[...] = jnp.dot(h1, w2_ref[...], preferred_element_type=jnp.float32) * dv


def _l2_body(g_ref, u_ref, dinv_ref, b2_ref, w3_ref, out_ref):
    dv = dinv_ref[...]
    h2 = jnp.maximum(dv * (g_ref[0] + g_ref[1] + u_ref[...]) + b2_ref[...], 0.0)
    out_ref[...] = jnp.dot(h2, w3_ref[...], preferred_element_type=jnp.float32) * dv


def _l3_body(g_ref, u_ref, dinv_ref, b3_ref, out_ref):
    dv = dinv_ref[...]
    out_ref[...] = dv * (g_ref[0] + g_ref[1] + u_ref[...]) + b3_ref[...]


def kernel(x, edge_index, W1, b1, W2, b2, W3, b3):
    N = x.shape[0]
    E = edge_index.shape[1]
    n_pad = (N // 128 + 1) * 128
    k_chunks = -(-E // (NW * CH))
    k_chunks = ((k_chunks + 2 * NB - 1) // (2 * NB)) * (2 * NB)
    e_pad = NW * CH * k_chunks
    blk = n_pad // 16
    grid = n_pad // blk

    f32 = jnp.float32
    src = jnp.concatenate([edge_index[0], jnp.zeros((e_pad - E,), jnp.int32)]).reshape(e_pad // CH, CH)
    dst = jnp.concatenate([edge_index[1], jnp.full((e_pad - E,), N, jnp.int32)]).reshape(e_pad // CH, CH)
    x8 = jnp.zeros((n_pad, 8), f32).at[:N, :3].set(x)
    ones8 = jnp.zeros((n_pad, 8), f32).at[:, 0].set(1.0)
    z8 = jnp.zeros((n_pad, 8), f32)
    z16 = jnp.zeros((n_pad, 16), f32)
    z32 = jnp.zeros((n_pad, 32), f32)
    W1p = jnp.zeros((8, 64), f32).at[:3].set(W1)
    b1r = b1.reshape(1, 64)
    b2r = b2.reshape(1, 32)
    b3r = b3.reshape(1, 16)

    agg8 = _agg_kernel(n_pad, 8, k_chunks)
    agg32 = _agg_kernel(n_pad, 32, k_chunks)
    agg16 = _agg_kernel(n_pad, 16, k_chunks)

    def parts(a, d):
        return a.reshape(2, n_pad, d)

    degp = parts(agg8(ones8, src, dst, z8), 8)

    dinv, u1 = pl.pallas_call(
        _prep_body,
        grid=(grid,),
        in_specs=[_gpart_spec(blk, 8), _rows_spec(blk, 8)],
        out_specs=[_rows_spec(blk, 1), _rows_spec(blk, 8)],
        out_shape=[
            jax.ShapeDtypeStruct((n_pad, 1), f32),
            jax.ShapeDtypeStruct((n_pad, 8), f32),
        ],
    )(degp, x8)

    g1 = parts(agg8(u1, src, dst, z8), 8)
    u2 = pl.pallas_call(
        _l1_body,
        grid=(grid,),
        in_specs=[
            _gpart_spec(blk, 8),
            _rows_spec(blk, 8),
            _rows_spec(blk, 1),
            _full_spec((8, 64)),
            _full_spec((1, 64)),
            _full_spec((64, 32)),
        ],
        out_specs=_rows_spec(blk, 32),
        out_shape=jax.ShapeDtypeStruct((n_pad, 32), f32),
    )(g1, u1, dinv, W1p, b1r, W2)

    g2 = parts(agg32(u2, src, dst, z32), 32)
    u3 = pl.pallas_call(
        _l2_body,
        grid=(grid,),
        in_specs=[
            _gpart_spec(blk, 32),
            _rows_spec(blk, 32),
            _rows_spec(blk, 1),
            _full_spec((1, 32)),
            _full_spec((32, 16)),
        ],
        out_specs=_rows_spec(blk, 16),
        out_shape=jax.ShapeDtypeStruct((n_pad, 16), f32),
    )(g2, u2, dinv, b2r, W3)

    g3 = parts(agg16(u3, src, dst, z16), 16)
    h3 = pl.pallas_call(
        _l3_body,
        grid=(grid,),
        in_specs=[
            _gpart_spec(blk, 16),
            _rows_spec(blk, 16),
            _rows_spec(blk, 1),
            _full_spec((1, 16)),
        ],
        out_specs=_rows_spec(blk, 16),
        out_shape=jax.ShapeDtypeStruct((n_pad, 16), f32),
    )(g3, u3, dinv, b3r)

    scores = _score_kernel(e_pad, k_chunks)(h3, src, dst)
    return scores[:E]


# speedup vs baseline: 24.4636x; 1.0900x over previous
import functools

import jax
import jax.numpy as jnp
from jax import lax
from jax.experimental import pallas as pl
from jax.experimental.pallas import tpu as pltpu
from jax.experimental.pallas import tpu_sc as plsc

NC = 2
NS = 16
NW = NC * NS
CH = 128
LN = 16
NB = 8

_SC_PARAMS = pltpu.CompilerParams(use_tc_tiling_on_sc=False, needs_layout_passes=False)


def _mesh():
    return plsc.VectorSubcoreMesh(
        core_axis_name="c", subcore_axis_name="s", num_cores=NC, num_subcores=NS
    )


@functools.cache
def _agg_kernel(n_pad, d, k_chunks, n_tab=1):
    rpt = n_pad // NS
    kb = k_chunks // NB
    nsb = kb // 2

    @functools.partial(
        pl.kernel,
        out_type=jax.ShapeDtypeStruct((n_tab * NC * n_pad, d), jnp.float32),
        mesh=_mesh(),
        scratch_types=[
            [pltpu.VMEM((NB, CH), jnp.int32) for _ in range(2)],
            [pltpu.VMEM((NB, CH), jnp.int32) for _ in range(2)],
            [pltpu.VMEM((CH, d), jnp.float32) for _ in range(NB)],
            [pltpu.VMEM((CH, d), jnp.float32) for _ in range(NB)],
            pltpu.VMEM_SHARED((n_pad, d), jnp.float32),
            pltpu.SemaphoreType.DMA,
            pltpu.SemaphoreType.DMA,
            pltpu.SemaphoreType.DMA,
        ],
        compiler_params=_SC_PARAMS,
    )
    def agg(*args):
        u_hbm = args[:n_tab]
        (src2_hbm, dst2_hbm, zero_hbm, out_hbm,
         sblk, dblk, rowsA, rowsB, acc, gsem, ssem, isem) = args[n_tab:]
        c = lax.axis_index("c")
        s = lax.axis_index("s")
        wid = s * NC + c
        base = wid * kb

        def load_idx(blk_i, slot):
            row0 = (base + blk_i) * NB
            a = pltpu.async_copy(src2_hbm.at[pl.ds(row0, NB)], sblk[slot], isem)
            b = pltpu.async_copy(dst2_hbm.at[pl.ds(row0, NB)], dblk[slot], isem)
            return a, b

        pltpu.sync_copy(zero_hbm.at[pl.ds(s * rpt, rpt)], acc.at[pl.ds(s * rpt, rpt)])
        plsc.subcore_barrier()

        for t in range(n_tab):
            for x in load_idx(0, 0):
                x.wait()

            def super_block(p, carry):
                i1 = 2 * p + 1
                i2 = jnp.where(2 * p + 2 < kb, 2 * p + 2, 0)
                la, lb = load_idx(i1, 1)
                gA = [pltpu.async_copy(u_hbm[t].at[sblk[0].at[b]], rowsA[b], gsem)
                      for b in range(NB)]
                for x in gA:
                    x.wait()
                sA = [pltpu.async_copy(rowsA[b], acc.at[dblk[0].at[b]], ssem, add=True)
                      for b in range(NB)]
                la.wait()
                lb.wait()
                gB = [pltpu.async_copy(u_hbm[t].at[sblk[1].at[b]], rowsB[b], gsem)
                      for b in range(NB)]
                for x in sA:
                    x.wait()
                lc, ld = load_idx(i2, 0)
                for x in gB:
                    x.wait()
                sB = [pltpu.async_copy(rowsB[b], acc.at[dblk[1].at[b]], ssem, add=True)
                      for b in range(NB)]
                lc.wait()
                ld.wait()
                for x in sB:
                    x.wait()
                return carry

            lax.fori_loop(0, nsb, super_block, 0)
            plsc.subcore_barrier()
            pltpu.sync_copy(
                acc.at[pl.ds(s * rpt, rpt)],
                out_hbm.at[pl.ds((t * NC + c) * n_pad + s * rpt, rpt)],
            )
            if t + 1 < n_tab:
                pltpu.sync_copy(zero_hbm.at[pl.ds(s * rpt, rpt)],
                                acc.at[pl.ds(s * rpt, rpt)])
                plsc.subcore_barrier()

    return agg


@functools.cache
def _deg_kernel(n_pad, k_chunks):
    d = 8
    rpt = n_pad // NS
    kb = k_chunks // NB
    nsb = kb // 2

    @functools.partial(
        pl.kernel,
        out_type=jax.ShapeDtypeStruct((NC * n_pad, d), jnp.float32),
        mesh=_mesh(),
        scratch_types=[
            [pltpu.VMEM((NB, CH), jnp.int32) for _ in range(2)],
            pltpu.VMEM((CH, d), jnp.float32),
            pltpu.VMEM_SHARED((n_pad, d), jnp.float32),
            pltpu.SemaphoreType.DMA,
            pltpu.SemaphoreType.DMA,
        ],
        compiler_params=_SC_PARAMS,
    )
    def deg(dst2_hbm, ones_hbm, zero_hbm, out_hbm, dblk, rows, acc, ssem, isem):
        c = lax.axis_index("c")
        s = lax.axis_index("s")
        wid = s * NC + c
        base = wid * kb
        pltpu.sync_copy(ones_hbm, rows)
        pltpu.sync_copy(zero_hbm.at[pl.ds(s * rpt, rpt)], acc.at[pl.ds(s * rpt, rpt)])
        plsc.subcore_barrier()

        def load_idx(blk_i, slot):
            row0 = (base + blk_i) * NB
            return pltpu.async_copy(dst2_hbm.at[pl.ds(row0, NB)], dblk[slot], isem)

        load_idx(0, 0).wait()

        def super_block(p, carry):
            i1 = 2 * p + 1
            i2 = jnp.where(2 * p + 2 < kb, 2 * p + 2, 0)
            la = load_idx(i1, 1)
            sA = [pltpu.async_copy(rows, acc.at[dblk[0].at[b]], ssem, add=True)
                  for b in range(NB)]
            la.wait()
            for x in sA:
                x.wait()
            lc = load_idx(i2, 0)
            sB = [pltpu.async_copy(rows, acc.at[dblk[1].at[b]], ssem, add=True)
                  for b in range(NB)]
            lc.wait()
            for x in sB:
                x.wait()
            return carry

        lax.fori_loop(0, nsb, super_block, 0)
        plsc.subcore_barrier()
        pltpu.sync_copy(acc.at[pl.ds(s * rpt, rpt)],
                        out_hbm.at[pl.ds(c * n_pad + s * rpt, rpt)])

    return deg


@functools.cache
def _score_kernel(e_pad, k_chunks):
    kb = k_chunks // NB
    nsb = kb // 2

    @functools.partial(
        pl.kernel,
        out_type=jax.ShapeDtypeStruct((e_pad,), jnp.float32),
        mesh=_mesh(),
        scratch_types=[
            [pltpu.VMEM((NB, CH), jnp.int32) for _ in range(2)],
            [pltpu.VMEM((NB, CH), jnp.int32) for _ in range(2)],
            [pltpu.VMEM((CH, LN), jnp.float32) for _ in range(NB)],
            [pltpu.VMEM((CH, LN), jnp.float32) for _ in range(NB)],
            [pltpu.VMEM((CH, LN), jnp.float32) for _ in range(NB)],
            [pltpu.VMEM((CH, LN), jnp.float32) for _ in range(NB)],
            pltpu.VMEM((NB * CH,), jnp.float32),
            pltpu.SemaphoreType.DMA,
            pltpu.SemaphoreType.DMA,
            pltpu.SemaphoreType.DMA,
        ],
        compiler_params=_SC_PARAMS,
    )
    def score(h_hbm, src2_hbm, dst2_hbm, out_hbm,
              sblk, dblk, arowsA, browsA, arowsB, browsB, svec,
              gsemA, gsemB, isem):
        c = lax.axis_index("c")
        s = lax.axis_index("s")
        wid = s * NC + c
        base = wid * kb

        def load_idx(blk_i, slot):
            row0 = (base + blk_i) * NB
            a = pltpu.async_copy(src2_hbm.at[pl.ds(row0, NB)], sblk[slot], isem)
            b = pltpu.async_copy(dst2_hbm.at[pl.ds(row0, NB)], dblk[slot], isem)
            return a, b

        def fire(slot, arows, brows, sem):
            g = [pltpu.async_copy(h_hbm.at[sblk[slot].at[b]], arows[b], sem)
                 for b in range(NB)]
            g += [pltpu.async_copy(h_hbm.at[dblk[slot].at[b]], brows[b], sem)
                  for b in range(NB)]
            return g

        lane = lax.iota(jnp.int32, LN)
        cols = [jnp.full((LN,), dd, jnp.int32) for dd in range(LN)]

        def compute(blk_i, arows, brows):
            for b in range(NB):
                for gg in range(CH // LN):
                    rid = lane + gg * LN
                    accs = [jnp.zeros((LN,), jnp.float32) for _ in range(4)]
                    for dd in range(LN):
                        accs[dd % 4] = accs[dd % 4] + (
                            plsc.load_gather(arows[b], [rid, cols[dd]])
                            * plsc.load_gather(brows[b], [rid, cols[dd]]))
                    acc_v = (accs[0] + accs[1]) + (accs[2] + accs[3])
                    svec[pl.ds(b * CH + gg * LN, LN)] = acc_v
            pltpu.sync_copy(svec, out_hbm.at[pl.ds((base + blk_i) * NB * CH, NB * CH)])

        for x in load_idx(0, 0):
            x.wait()

        def super_block(p, carry):
            i1 = 2 * p + 1
            i2 = jnp.where(2 * p + 2 < kb, 2 * p + 2, 0)
            la, lb = load_idx(i1, 1)
            gA = fire(0, arowsA, browsA, gsemA)
            la.wait()
            lb.wait()
            gB = fire(1, arowsB, browsB, gsemB)
            for x in gA:
                x.wait()
            compute(2 * p, arowsA, browsA)
            lc, ld = load_idx(i2, 0)
            for x in gB:
                x.wait()
            compute(i1, arowsB, browsB)
            lc.wait()
            ld.wait()
            return carry

        lax.fori_loop(0, nsb, super_block, 0)

    return score


def _rows_spec(blk, d):
    return pl.BlockSpec((blk, d), lambda i: (i, 0))


def _gpart_spec(blk, d):
    return pl.BlockSpec((2, blk, d), lambda i: (0, i, 0))


def _full_spec(shape):
    return pl.BlockSpec(shape, lambda i: tuple(0 for _ in shape))


def _prep_body(degp_ref, x_ref, dinv_ref, u1_ref):
    deg = degp_ref[0, :, 0:1] + degp_ref[1, :, 0:1] + 1.0
    dv = lax.rsqrt(deg)
    dinv_ref[...] = dv
    u1_ref[...] = x_ref[...] * dv


def _l1_body(g_ref, u_ref, dinv_ref, w1_ref, b1_ref, w2_ref, ua_ref, ub_ref):
    dv = dinv_ref[...]
    s1 = dv * (g_ref[0] + g_ref[1] + u_ref[...])
    h1 = jnp.maximum(
        jnp.dot(s1, w1_ref[...], preferred_element_type=jnp.float32) + b1_ref[...], 0.0
    )
    u2 = jnp.dot(h1, w2_ref[...], preferred_element_type=jnp.float32) * dv
    ua_ref[...] = u2[:, :16]
    ub_ref[...] = u2[:, 16:]


def _l2_body(ga_ref, gb_ref, ua_ref, ub_ref, dinv_ref, b2a_ref, b2b_ref,
             w3a_ref, w3b_ref, out_ref):
    dv = dinv_ref[...]
    h2a = jnp.maximum(dv * (ga_ref[0] + ga_ref[1] + ua_ref[...]) + b2a_ref[...], 0.0)
    h2b = jnp.maximum(dv * (gb_ref[0] + gb_ref[1] + ub_ref[...]) + b2b_ref[...], 0.0)
    u3 = (jnp.dot(h2a, w3a_ref[...], preferred_element_type=jnp.float32)
          + jnp.dot(h2b, w3b_ref[...], preferred_element_type=jnp.float32))
    out_ref[...] = u3 * dv


def _l3_body(g_ref, u_ref, dinv_ref, b3_ref, out_ref):
    dv = dinv_ref[...]
    out_ref[...] = dv * (g_ref[0] + g_ref[1] + u_ref[...]) + b3_ref[...]


def kernel(x, edge_index, W1, b1, W2, b2, W3, b3):
    N = x.shape[0]
    E = edge_index.shape[1]
    n_pad = (N // 128 + 1) * 128
    k_chunks = -(-E // (NW * CH))
    k_chunks = ((k_chunks + 2 * NB - 1) // (2 * NB)) * (2 * NB)
    e_pad = NW * CH * k_chunks
    blk = n_pad // 16
    grid = n_pad // blk

    f32 = jnp.float32
    src = jnp.concatenate([edge_index[0], jnp.zeros((e_pad - E,), jnp.int32)]).reshape(e_pad // CH, CH)
    dst = jnp.concatenate([edge_index[1], jnp.full((e_pad - E,), N, jnp.int32)]).reshape(e_pad // CH, CH)
    x8 = jnp.zeros((n_pad, 8), f32).at[:N, :3].set(x)
    ones_ch = jnp.zeros((CH, 8), f32).at[:, 0].set(1.0)
    z8 = jnp.zeros((n_pad, 8), f32)
    z16 = jnp.zeros((n_pad, 16), f32)
    W1p = jnp.zeros((8, 64), f32).at[:3].set(W1)
    b1r = b1.reshape(1, 64)
    b2a = b2[:16].reshape(1, 16)
    b2b = b2[16:].reshape(1, 16)
    b3r = b3.reshape(1, 16)
    W3a = W3[:16]
    W3b = W3[16:]

    agg8 = _agg_kernel(n_pad, 8, k_chunks)
    agg16x2 = _agg_kernel(n_pad, 16, k_chunks, n_tab=2)
    agg16 = _agg_kernel(n_pad, 16, k_chunks)

    degp = _deg_kernel(n_pad, k_chunks)(dst, ones_ch, z8).reshape(2, n_pad, 8)

    dinv, u1 = pl.pallas_call(
        _prep_body,
        grid=(grid,),
        in_specs=[_gpart_spec(blk, 8), _rows_spec(blk, 8)],
        out_specs=[_rows_spec(blk, 1), _rows_spec(blk, 8)],
        out_shape=[
            jax.ShapeDtypeStruct((n_pad, 1), f32),
            jax.ShapeDtypeStruct((n_pad, 8), f32),
        ],
    )(degp, x8)

    g1 = agg8(u1, src, dst, z8).reshape(2, n_pad, 8)
    u2a, u2b = pl.pallas_call(
        _l1_body,
        grid=(grid,),
        in_specs=[
            _gpart_spec(blk, 8),
            _rows_spec(blk, 8),
            _rows_spec(blk, 1),
            _full_spec((8, 64)),
            _full_spec((1, 64)),
            _full_spec((64, 32)),
        ],
        out_specs=[_rows_spec(blk, 16), _rows_spec(blk, 16)],
        out_shape=[
            jax.ShapeDtypeStruct((n_pad, 16), f32),
            jax.ShapeDtypeStruct((n_pad, 16), f32),
        ],
    )(g1, u1, dinv, W1p, b1r, W2)

    g2 = agg16x2(u2a, u2b, src, dst, z16).reshape(2, 2, n_pad, 16)
    u3 = pl.pallas_call(
        _l2_body,
        grid=(grid,),
        in_specs=[
            _gpart_spec(blk, 16),
            _gpart_spec(blk, 16),
            _rows_spec(blk, 16),
            _rows_spec(blk, 16),
            _rows_spec(blk, 1),
            _full_spec((1, 16)),
            _full_spec((1, 16)),
            _full_spec((16, 16)),
            _full_spec((16, 16)),
        ],
        out_specs=_rows_spec(blk, 16),
        out_shape=jax.ShapeDtypeStruct((n_pad, 16), f32),
    )(g2[0], g2[1], u2a, u2b, dinv, b2a, b2b, W3a, W3b)

    g3 = agg16(u3, src, dst, z16).reshape(2, n_pad, 16)
    h3 = pl.pallas_call(
        _l3_body,
        grid=(grid,),
        in_specs=[
            _gpart_spec(blk, 16),
            _rows_spec(blk, 16),
            _rows_spec(blk, 1),
            _full_spec((1, 16)),
        ],
        out_specs=_rows_spec(blk, 16),
        out_shape=jax.ShapeDtypeStruct((n_pad, 16), f32),
    )(g3, u3, dinv, b3r)

    scores = _score_kernel(e_pad, k_chunks)(h3, src, dst)
    return scores[:E]


# speedup vs baseline: 25.7077x; 1.0509x over previous
import functools

import jax
import jax.numpy as jnp
from jax import lax
from jax.experimental import pallas as pl
from jax.experimental.pallas import tpu as pltpu
from jax.experimental.pallas import tpu_sc as plsc

NC = 2
NS = 16
NW = NC * NS
CH = 128
LN = 16
NB = 8

_SC_PARAMS = pltpu.CompilerParams(use_tc_tiling_on_sc=False, needs_layout_passes=False)


def _mesh():
    return plsc.VectorSubcoreMesh(
        core_axis_name="c", subcore_axis_name="s", num_cores=NC, num_subcores=NS
    )


@functools.cache
def _agg_kernel(n_pad, d, k_chunks, n_tab=1):
    rpt = n_pad // NS
    kb = k_chunks // NB
    nsb = kb // 2

    @functools.partial(
        pl.kernel,
        out_type=jax.ShapeDtypeStruct((n_tab * NC * n_pad, d), jnp.float32),
        mesh=_mesh(),
        scratch_types=[
            [pltpu.VMEM((NB, CH), jnp.int32) for _ in range(2)],
            [pltpu.VMEM((NB, CH), jnp.int32) for _ in range(2)],
            [pltpu.VMEM((CH, d), jnp.float32) for _ in range(NB)],
            [pltpu.VMEM((CH, d), jnp.float32) for _ in range(NB)],
            pltpu.VMEM_SHARED((n_pad, d), jnp.float32),
            pltpu.SemaphoreType.DMA,
            pltpu.SemaphoreType.DMA,
            pltpu.SemaphoreType.DMA,
        ],
        compiler_params=_SC_PARAMS,
    )
    def agg(*args):
        u_hbm = args[:n_tab]
        (src2_hbm, dst2_hbm, zero_hbm, out_hbm,
         sblk, dblk, rowsA, rowsB, acc, gsem, ssem, isem) = args[n_tab:]
        c = lax.axis_index("c")
        s = lax.axis_index("s")
        wid = s * NC + c
        base = wid * kb

        def load_idx(blk_i, slot):
            row0 = (base + blk_i) * NB
            a = pltpu.async_copy(src2_hbm.at[pl.ds(row0, NB)], sblk[slot], isem)
            b = pltpu.async_copy(dst2_hbm.at[pl.ds(row0, NB)], dblk[slot], isem)
            return a, b

        pltpu.sync_copy(zero_hbm.at[pl.ds(s * rpt, rpt)], acc.at[pl.ds(s * rpt, rpt)])
        plsc.subcore_barrier()

        for t in range(n_tab):
            for x in load_idx(0, 0):
                x.wait()

            def super_block(p, carry):
                i1 = 2 * p + 1
                i2 = jnp.where(2 * p + 2 < kb, 2 * p + 2, 0)
                la, lb = load_idx(i1, 1)
                gA = [pltpu.async_copy(u_hbm[t].at[sblk[0].at[b]], rowsA[b], gsem)
                      for b in range(NB)]
                for x in gA:
                    x.wait()
                sA = [pltpu.async_copy(rowsA[b], acc.at[dblk[0].at[b]], ssem, add=True)
                      for b in range(NB)]
                la.wait()
                lb.wait()
                gB = [pltpu.async_copy(u_hbm[t].at[sblk[1].at[b]], rowsB[b], gsem)
                      for b in range(NB)]
                for x in sA:
                    x.wait()
                lc, ld = load_idx(i2, 0)
                for x in gB:
                    x.wait()
                sB = [pltpu.async_copy(rowsB[b], acc.at[dblk[1].at[b]], ssem, add=True)
                      for b in range(NB)]
                lc.wait()
                ld.wait()
                for x in sB:
                    x.wait()
                return carry

            lax.fori_loop(0, nsb, super_block, 0)
            plsc.subcore_barrier()
            pltpu.sync_copy(
                acc.at[pl.ds(s * rpt, rpt)],
                out_hbm.at[pl.ds((t * NC + c) * n_pad + s * rpt, rpt)],
            )
            if t + 1 < n_tab:
                pltpu.sync_copy(zero_hbm.at[pl.ds(s * rpt, rpt)],
                                acc.at[pl.ds(s * rpt, rpt)])
                plsc.subcore_barrier()

    return agg


@functools.cache
def _deg_kernel(n_pad, k_chunks):
    d = 8
    rpt = n_pad // NS
    kb = k_chunks // NB
    nsb = kb // 2

    @functools.partial(
        pl.kernel,
        out_type=jax.ShapeDtypeStruct((NC * n_pad, d), jnp.float32),
        mesh=_mesh(),
        scratch_types=[
            [pltpu.VMEM((NB, CH), jnp.int32) for _ in range(2)],
            pltpu.VMEM((CH, d), jnp.float32),
            pltpu.VMEM_SHARED((n_pad, d), jnp.float32),
            pltpu.SemaphoreType.DMA,
            pltpu.SemaphoreType.DMA,
        ],
        compiler_params=_SC_PARAMS,
    )
    def deg(dst2_hbm, ones_hbm, zero_hbm, out_hbm, dblk, rows, acc, ssem, isem):
        c = lax.axis_index("c")
        s = lax.axis_index("s")
        wid = s * NC + c
        base = wid * kb
        pltpu.sync_copy(ones_hbm, rows)
        pltpu.sync_copy(zero_hbm.at[pl.ds(s * rpt, rpt)], acc.at[pl.ds(s * rpt, rpt)])
        plsc.subcore_barrier()

        def load_idx(blk_i, slot):
            row0 = (base + blk_i) * NB
            return pltpu.async_copy(dst2_hbm.at[pl.ds(row0, NB)], dblk[slot], isem)

        load_idx(0, 0).wait()

        def super_block(p, carry):
            i1 = 2 * p + 1
            i2 = jnp.where(2 * p + 2 < kb, 2 * p + 2, 0)
            la = load_idx(i1, 1)
            sA = [pltpu.async_copy(rows, acc.at[dblk[0].at[b]], ssem, add=True)
                  for b in range(NB)]
            la.wait()
            for x in sA:
                x.wait()
            lc = load_idx(i2, 0)
            sB = [pltpu.async_copy(rows, acc.at[dblk[1].at[b]], ssem, add=True)
                  for b in range(NB)]
            lc.wait()
            for x in sB:
                x.wait()
            return carry

        lax.fori_loop(0, nsb, super_block, 0)
        plsc.subcore_barrier()
        pltpu.sync_copy(acc.at[pl.ds(s * rpt, rpt)],
                        out_hbm.at[pl.ds(c * n_pad + s * rpt, rpt)])

    return deg


@functools.cache
def _score_kernel(e_pad, k_chunks):
    kb = k_chunks // NB
    nsb = kb // 2

    @functools.partial(
        pl.kernel,
        out_type=jax.ShapeDtypeStruct((e_pad,), jnp.float32),
        mesh=_mesh(),
        scratch_types=[
            [pltpu.VMEM((NB, CH), jnp.int32) for _ in range(2)],
            [pltpu.VMEM((NB, CH), jnp.int32) for _ in range(2)],
            [pltpu.VMEM((CH, LN), jnp.float32) for _ in range(NB)],
            [pltpu.VMEM((CH, LN), jnp.float32) for _ in range(NB)],
            [pltpu.VMEM((CH, LN), jnp.float32) for _ in range(NB)],
            [pltpu.VMEM((CH, LN), jnp.float32) for _ in range(NB)],
            pltpu.VMEM((NB * CH,), jnp.float32),
            pltpu.SemaphoreType.DMA,
            pltpu.SemaphoreType.DMA,
            pltpu.SemaphoreType.DMA,
        ],
        compiler_params=_SC_PARAMS,
    )
    def score(h_hbm, src2_hbm, dst2_hbm, out_hbm,
              sblk, dblk, arowsA, browsA, arowsB, browsB, svec,
              gsemA, gsemB, isem):
        c = lax.axis_index("c")
        s = lax.axis_index("s")
        wid = s * NC + c
        base = wid * kb

        def load_idx(blk_i, slot):
            row0 = (base + blk_i) * NB
            a = pltpu.async_copy(src2_hbm.at[pl.ds(row0, NB)], sblk[slot], isem)
            b = pltpu.async_copy(dst2_hbm.at[pl.ds(row0, NB)], dblk[slot], isem)
            return a, b

        def fire(slot, arows, brows, sem):
            g = [pltpu.async_copy(h_hbm.at[sblk[slot].at[b]], arows[b], sem)
                 for b in range(NB)]
            g += [pltpu.async_copy(h_hbm.at[dblk[slot].at[b]], brows[b], sem)
                  for b in range(NB)]
            return g

        lane = lax.iota(jnp.int32, LN)
        cols = [lax.rem(lane + dd, jnp.full((LN,), LN, jnp.int32)) for dd in range(LN)]

        def compute(blk_i, arows, brows):
            for b in range(NB):
                def ggbody(gg, carry, b=b):
                    rid = lane + gg * LN
                    accs = [jnp.zeros((LN,), jnp.float32) for _ in range(4)]
                    for dd in range(LN):
                        accs[dd % 4] = accs[dd % 4] + (
                            plsc.load_gather(arows[b], [rid, cols[dd]])
                            * plsc.load_gather(brows[b], [rid, cols[dd]]))
                    acc_v = (accs[0] + accs[1]) + (accs[2] + accs[3])
                    svec[pl.ds(b * CH + gg * LN, LN)] = acc_v
                    return carry

                lax.fori_loop(0, CH // LN, ggbody, 0)
            pltpu.sync_copy(svec, out_hbm.at[pl.ds((base + blk_i) * NB * CH, NB * CH)])

        for x in load_idx(0, 0):
            x.wait()

        def super_block(p, carry):
            i1 = 2 * p + 1
            i2 = jnp.where(2 * p + 2 < kb, 2 * p + 2, 0)
            la, lb = load_idx(i1, 1)
            gA = fire(0, arowsA, browsA, gsemA)
            la.wait()
            lb.wait()
            gB = fire(1, arowsB, browsB, gsemB)
            for x in gA:
                x.wait()
            compute(2 * p, arowsA, browsA)
            lc, ld = load_idx(i2, 0)
            for x in gB:
                x.wait()
            compute(i1, arowsB, browsB)
            lc.wait()
            ld.wait()
            return carry

        lax.fori_loop(0, nsb, super_block, 0)

    return score


def _rows_spec(blk, d):
    return pl.BlockSpec((blk, d), lambda i: (i, 0))


def _gpart_spec(blk, d):
    return pl.BlockSpec((2, blk, d), lambda i: (0, i, 0))


def _full_spec(shape):
    return pl.BlockSpec(shape, lambda i: tuple(0 for _ in shape))


def _prep_body(degp_ref, x_ref, dinv_ref, u1_ref):
    deg = degp_ref[0, :, 0:1] + degp_ref[1, :, 0:1] + 1.0
    dv = lax.rsqrt(deg)
    dinv_ref[...] = dv
    u1_ref[...] = x_ref[...] * dv


def _l1_body(g_ref, u_ref, dinv_ref, w1_ref, b1_ref, w2_ref, ua_ref, ub_ref):
    dv = dinv_ref[...]
    s1 = dv * (g_ref[0] + g_ref[1] + u_ref[...])
    h1 = jnp.maximum(
        jnp.dot(s1, w1_ref[...], preferred_element_type=jnp.float32) + b1_ref[...], 0.0
    )
    u2 = jnp.dot(h1, w2_ref[...], preferred_element_type=jnp.float32) * dv
    ua_ref[...] = u2[:, :16]
    ub_ref[...] = u2[:, 16:]


def _l2_body(ga_ref, gb_ref, ua_ref, ub_ref, dinv_ref, b2a_ref, b2b_ref,
             w3a_ref, w3b_ref, out_ref):
    dv = dinv_ref[...]
    h2a = jnp.maximum(dv * (ga_ref[0] + ga_ref[1] + ua_ref[...]) + b2a_ref[...], 0.0)
    h2b = jnp.maximum(dv * (gb_ref[0] + gb_ref[1] + ub_ref[...]) + b2b_ref[...], 0.0)
    u3 = (jnp.dot(h2a, w3a_ref[...], preferred_element_type=jnp.float32)
          + jnp.dot(h2b, w3b_ref[...], preferred_element_type=jnp.float32))
    out_ref[...] = u3 * dv


def _l3_body(g_ref, u_ref, dinv_ref, b3_ref, out_ref):
    dv = dinv_ref[...]
    out_ref[...] = dv * (g_ref[0] + g_ref[1] + u_ref[...]) + b3_ref[...]


def kernel(x, edge_index, W1, b1, W2, b2, W3, b3):
    N = x.shape[0]
    E = edge_index.shape[1]
    n_pad = (N // 128 + 1) * 128
    k_chunks = -(-E // (NW * CH))
    k_chunks = ((k_chunks + 2 * NB - 1) // (2 * NB)) * (2 * NB)
    e_pad = NW * CH * k_chunks
    blk = n_pad // 16
    grid = n_pad // blk

    f32 = jnp.float32
    src = jnp.concatenate([edge_index[0], jnp.zeros((e_pad - E,), jnp.int32)]).reshape(e_pad // CH, CH)
    dst = jnp.concatenate([edge_index[1], jnp.full((e_pad - E,), N, jnp.int32)]).reshape(e_pad // CH, CH)
    x8 = jnp.zeros((n_pad, 8), f32).at[:N, :3].set(x)
    ones_ch = jnp.zeros((CH, 8), f32).at[:, 0].set(1.0)
    z8 = jnp.zeros((n_pad, 8), f32)
    z16 = jnp.zeros((n_pad, 16), f32)
    W1p = jnp.zeros((8, 64), f32).at[:3].set(W1)
    b1r = b1.reshape(1, 64)
    b2a = b2[:16].reshape(1, 16)
    b2b = b2[16:].reshape(1, 16)
    b3r = b3.reshape(1, 16)
    W3a = W3[:16]
    W3b = W3[16:]

    agg8 = _agg_kernel(n_pad, 8, k_chunks)
    agg16x2 = _agg_kernel(n_pad, 16, k_chunks, n_tab=2)
    agg16 = _agg_kernel(n_pad, 16, k_chunks)

    degp = _deg_kernel(n_pad, k_chunks)(dst, ones_ch, z8).reshape(2, n_pad, 8)

    dinv, u1 = pl.pallas_call(
        _prep_body,
        grid=(grid,),
        in_specs=[_gpart_spec(blk, 8), _rows_spec(blk, 8)],
        out_specs=[_rows_spec(blk, 1), _rows_spec(blk, 8)],
        out_shape=[
            jax.ShapeDtypeStruct((n_pad, 1), f32),
            jax.ShapeDtypeStruct((n_pad, 8), f32),
        ],
    )(degp, x8)

    g1 = agg8(u1, src, dst, z8).reshape(2, n_pad, 8)
    u2a, u2b = pl.pallas_call(
        _l1_body,
        grid=(grid,),
        in_specs=[
            _gpart_spec(blk, 8),
            _rows_spec(blk, 8),
            _rows_spec(blk, 1),
            _full_spec((8, 64)),
            _full_spec((1, 64)),
            _full_spec((64, 32)),
        ],
        out_specs=[_rows_spec(blk, 16), _rows_spec(blk, 16)],
        out_shape=[
            jax.ShapeDtypeStruct((n_pad, 16), f32),
            jax.ShapeDtypeStruct((n_pad, 16), f32),
        ],
    )(g1, u1, dinv, W1p, b1r, W2)

    g2 = agg16x2(u2a, u2b, src, dst, z16).reshape(2, 2, n_pad, 16)
    u3 = pl.pallas_call(
        _l2_body,
        grid=(grid,),
        in_specs=[
            _gpart_spec(blk, 16),
            _gpart_spec(blk, 16),
            _rows_spec(blk, 16),
            _rows_spec(blk, 16),
            _rows_spec(blk, 1),
            _full_spec((1, 16)),
            _full_spec((1, 16)),
            _full_spec((16, 16)),
            _full_spec((16, 16)),
        ],
        out_specs=_rows_spec(blk, 16),
        out_shape=jax.ShapeDtypeStruct((n_pad, 16), f32),
    )(g2[0], g2[1], u2a, u2b, dinv, b2a, b2b, W3a, W3b)

    g3 = agg16(u3, src, dst, z16).reshape(2, n_pad, 16)
    h3 = pl.pallas_call(
        _l3_body,
        grid=(grid,),
        in_specs=[
            _gpart_spec(blk, 16),
            _rows_spec(blk, 16),
            _rows_spec(blk, 1),
            _full_spec((1, 16)),
        ],
        out_specs=_rows_spec(blk, 16),
        out_shape=jax.ShapeDtypeStruct((n_pad, 16), f32),
    )(g3, u3, dinv, b3r)

    scores = _score_kernel(e_pad, k_chunks)(h3, src, dst)
    return scores[:E]
